# baseline (speedup 1.0000x reference)
import functools

import jax
import jax.numpy as jnp
from jax import lax
from jax.experimental import pallas as pl
from jax.experimental.pallas import tpu as pltpu
from jax.experimental.pallas import tpu_sc as plsc

N = 10000
E = 320000
D = 128
D_EDGE = 16

NC = 2
NS = 16
LANES = 16

C = 40
EPT = E // (NC * NS)
NCH = EPT // C
PASS_NCH = (124, 126)
SLAB = 126 * C
RQ = 624
TAIL = N - NS * RQ


def _edge_linear_both(ea_T, W0, b0, W1, b1):
    BE = 16000
    dn = (((0,), (0,)), ((), ()))

    def body(ea_ref, w0_ref, b0_ref, w1_ref, b1_ref, o0_ref, o1_ref):
        ea = ea_ref[...]
        o0_ref[...] = (
            lax.dot_general(ea, w0_ref[...], dn, preferred_element_type=jnp.float32)
            + b0_ref[...]
        )
        o1_ref[...] = (
            lax.dot_general(ea, w1_ref[...], dn, preferred_element_type=jnp.float32)
            + b1_ref[...]
        )

    return pl.pallas_call(
        body,
        grid=(E // BE,),
        in_specs=[
            pl.BlockSpec((D_EDGE, BE), lambda i: (0, i)),
            pl.BlockSpec((D_EDGE, D), lambda i: (0, 0)),
            pl.BlockSpec((1, D), lambda i: (0, 0)),
            pl.BlockSpec((D_EDGE, D), lambda i: (0, 0)),
            pl.BlockSpec((1, D), lambda i: (0, 0)),
        ],
        out_specs=[
            pl.BlockSpec((BE, D), lambda i: (i, 0)),
            pl.BlockSpec((BE, D), lambda i: (i, 0)),
        ],
        out_shape=[
            jax.ShapeDtypeStruct((E, D), jnp.float32),
            jax.ShapeDtypeStruct((E, D), jnp.float32),
        ],
        compiler_params=pltpu.CompilerParams(
            dimension_semantics=("parallel",)
        ),
    )(ea_T, W0, b0.reshape(1, D), W1, b1.reshape(1, D))


def _sc_partial_agg(h, e, src, dst):
    mesh = plsc.VectorSubcoreMesh(
        core_axis_name="c", subcore_axis_name="s", num_cores=NC, num_subcores=NS
    )

    @functools.partial(
        pl.kernel,
        out_type=jax.ShapeDtypeStruct((NC, N, D), jnp.float32),
        mesh=mesh,
        scratch_types=[
            pltpu.VMEM((SLAB,), jnp.int32),
            pltpu.VMEM((SLAB,), jnp.int32),
            pltpu.VMEM((C, D), jnp.float32),
            pltpu.VMEM((C, D), jnp.float32),
            pltpu.VMEM((C, D), jnp.float32),
            pltpu.VMEM((C, D), jnp.float32),
            pltpu.VMEM((C, D), jnp.float32),
            pltpu.VMEM((C, D), jnp.float32),
            pltpu.VMEM((C,), jnp.int32),
            pltpu.VMEM((C,), jnp.int32),
            pltpu.VMEM_SHARED((N, D), jnp.float32),
            pltpu.SemaphoreType.DMA,
            pltpu.SemaphoreType.DMA,
            pltpu.SemaphoreType.DMA,
            pltpu.SemaphoreType.DMA,
            pltpu.SemaphoreType.DMA,
            pltpu.SemaphoreType.DMA,
            pltpu.SemaphoreType.DMA,
        ],
    )
    def k(h_hbm, e_hbm, src_hbm, dst_hbm, out_hbm, src_v, dst_v,
          g0, g1, e0, e1, m0, m1, du0, du1, agg_s,
          isem, gs0, gs1, es0, es1, ss0, ss1):
        cid = lax.axis_index("c")
        sid = lax.axis_index("s")
        wid = sid * NC + cid
        base_e = wid * EPT

        bufs = ((g0, e0, m0, du0, gs0, es0, ss0),
                (g1, e1, m1, du1, gs1, es1, ss1))

        def load_slabs(eoff, n_edges):
            pltpu.async_copy(
                src_hbm.at[pl.ds(eoff, n_edges)], src_v.at[pl.ds(0, n_edges)], isem
            )
            pltpu.async_copy(
                dst_hbm.at[pl.ds(eoff, n_edges)], dst_v.at[pl.ds(0, n_edges)], isem
            )

        def wait_slabs(eoff, n_edges):
            pltpu.make_async_copy(
                src_hbm.at[pl.ds(eoff, n_edges)], src_v.at[pl.ds(0, n_edges)], isem
            ).wait()
            pltpu.make_async_copy(
                dst_hbm.at[pl.ds(eoff, n_edges)], dst_v.at[pl.ds(0, n_edges)], isem
            ).wait()

        load_slabs(base_e, PASS_NCH[0] * C)

        @pl.loop(0, C)
        def _(r):
            for j in range(0, D, LANES):
                m0[r, pl.ds(j, LANES)] = jnp.zeros((LANES,), jnp.float32)

        @pl.loop(0, RQ - C + 1, step=C)
        def _(r0):
            pltpu.sync_copy(m0, agg_s.at[pl.ds(sid * RQ + r0, C)])

        pltpu.sync_copy(
            m0.at[pl.ds(0, RQ - (RQ // C) * C)],
            agg_s.at[pl.ds(sid * RQ + (RQ // C) * C, RQ - (RQ // C) * C)],
        )

        @pl.when(sid == 0)
        def _():
            pltpu.sync_copy(m0.at[pl.ds(0, TAIL)], agg_s.at[pl.ds(NS * RQ, TAIL)])

        wait_slabs(base_e, PASS_NCH[0] * C)
        plsc.subcore_barrier()

        def wait_ge(b):
            g, ev, _, _, gs, es, _ = bufs[b]
            pltpu.make_async_copy(h_hbm.at[pl.ds(0, C)], g, gs).wait()
            pltpu.make_async_copy(h_hbm.at[pl.ds(0, C)], ev, es).wait()

        def wait_s(b):
            _, _, m, _, _, _, ss = bufs[b]
            pltpu.make_async_copy(h_hbm.at[pl.ds(0, C)], m, ss).wait()

        def compute(b):
            g, ev, m, _, _, _, _ = bufs[b]

            @plsc.parallel_loop(0, C, step=1, unroll=4)
            def _(r):
                for j in range(0, D, LANES):
                    m[r, pl.ds(j, LANES)] = jnp.maximum(
                        g[r, pl.ds(j, LANES)] + ev[r, pl.ds(j, LANES)], 0.0
                    )

        def run_pass(eoff, nch):
            def issue(i, b):
                g, ev, _, _, gs, es, _ = bufs[b]
                pltpu.async_copy(h_hbm.at[src_v.at[pl.ds(i * C, C)]], g, gs)
                pltpu.async_copy(e_hbm.at[pl.ds(eoff + i * C, C)], ev, es)

            def scatter(i, b):
                _, _, m, du, _, _, ss = bufs[b]
                for kk in (0, 16, 24):
                    du[pl.ds(kk, LANES)] = dst_v[pl.ds(i * C + kk, LANES)]
                pltpu.async_copy(m, agg_s.at[du], ss, add=True)

            issue(0, 0)
            issue(1, 1)

            wait_ge(0)
            compute(0)
            issue(2, 0)
            scatter(0, 0)
            wait_ge(1)
            compute(1)
            issue(3, 1)
            scatter(1, 1)

            @pl.loop(2, nch - 2, step=2)
            def _(i):
                wait_ge(0)
                wait_s(0)
                compute(0)
                issue(i + 2, 0)
                scatter(i, 0)
                wait_ge(1)
                wait_s(1)
                compute(1)
                issue(i + 3, 1)
                scatter(i + 1, 1)

            wait_ge(0)
            wait_s(0)
            compute(0)
            scatter(nch - 2, 0)
            wait_ge(1)
            wait_s(1)
            compute(1)
            scatter(nch - 1, 1)
            wait_s(0)
            wait_s(1)

        run_pass(base_e, PASS_NCH[0])
        load_slabs(base_e + PASS_NCH[0] * C, PASS_NCH[1] * C)
        wait_slabs(base_e + PASS_NCH[0] * C, PASS_NCH[1] * C)
        run_pass(base_e + PASS_NCH[0] * C, PASS_NCH[1])

        plsc.subcore_barrier()
        pltpu.sync_copy(
            agg_s.at[pl.ds(sid * RQ, RQ)],
            out_hbm.at[cid, pl.ds(sid * RQ, RQ)],
        )

        @pl.when(sid == 0)
        def _():
            pltpu.sync_copy(
                agg_s.at[pl.ds(NS * RQ, TAIL)],
                out_hbm.at[cid, pl.ds(NS * RQ, TAIL)],
            )

    return k(h, e, src, dst)


def _mlp_layer(h, p, W1, b1, W2, b2):
    BR = 2000

    def body(h_ref, p_ref, w1, b1r, w2, b2r, o_ref):
        z = h_ref[...] + p_ref[0] + p_ref[1]
        t = jnp.maximum(
            jnp.dot(z, w1[...], preferred_element_type=jnp.float32) + b1r[...], 0.0
        )
        o_ref[...] = jnp.maximum(
            jnp.dot(t, w2[...], preferred_element_type=jnp.float32) + b2r[...], 0.0
        )

    return pl.pallas_call(
        body,
        grid=(N // BR,),
        in_specs=[
            pl.BlockSpec((BR, D), lambda i: (i, 0)),
            pl.BlockSpec((NC, BR, D), lambda i: (0, i, 0)),
            pl.BlockSpec((D, D), lambda i: (0, 0)),
            pl.BlockSpec((1, D), lambda i: (0, 0)),
            pl.BlockSpec((D, D), lambda i: (0, 0)),
            pl.BlockSpec((1, D), lambda i: (0, 0)),
        ],
        out_specs=pl.BlockSpec((BR, D), lambda i: (i, 0)),
        out_shape=jax.ShapeDtypeStruct((N, D), jnp.float32),
    )(h, p, W1, b1.reshape(1, D), W2, b2.reshape(1, D))


def _mlp_final(h, p, W1, b1, W2, b2, fc1_W, fc1_b, fc2_W, fc2_b):
    BR = 2000

    def body(h_ref, p_ref, w1, b1r, w2, b2r, f1, f1b, f2, f2b, o_ref):
        z = h_ref[...] + p_ref[0] + p_ref[1]
        t = jnp.maximum(
            jnp.dot(z, w1[...], preferred_element_type=jnp.float32) + b1r[...], 0.0
        )
        h2 = jnp.maximum(
            jnp.dot(t, w2[...], preferred_element_type=jnp.float32) + b2r[...], 0.0
        )
        t2 = jnp.maximum(
            jnp.dot(h2, f1[...], preferred_element_type=jnp.float32) + f1b[...], 0.0
        )
        o_ref[...] = (
            jnp.dot(t2, f2[...], preferred_element_type=jnp.float32) + f2b[...]
        )

    return pl.pallas_call(
        body,
        grid=(N // BR,),
        in_specs=[
            pl.BlockSpec((BR, D), lambda i: (i, 0)),
            pl.BlockSpec((NC, BR, D), lambda i: (0, i, 0)),
            pl.BlockSpec((D, D), lambda i: (0, 0)),
            pl.BlockSpec((1, D), lambda i: (0, 0)),
            pl.BlockSpec((D, D), lambda i: (0, 0)),
            pl.BlockSpec((1, D), lambda i: (0, 0)),
            pl.BlockSpec((D, D), lambda i: (0, 0)),
            pl.BlockSpec((1, D), lambda i: (0, 0)),
            pl.BlockSpec((D, D), lambda i: (0, 0)),
            pl.BlockSpec((1, D), lambda i: (0, 0)),
        ],
        out_specs=pl.BlockSpec((BR, D), lambda i: (i, 0)),
        out_shape=jax.ShapeDtypeStruct((N, D), jnp.float32),
    )(h, p, W1, b1.reshape(1, D), W2, b2.reshape(1, D),
      fc1_W, fc1_b.reshape(1, D), fc2_W, fc2_b.reshape(1, D))


def kernel(x, edge_index, edge_attr,
           lin0_W, lin0_b, mlp0_W1, mlp0_b1, mlp0_W2, mlp0_b2,
           lin1_W, lin1_b, mlp1_W1, mlp1_b1, mlp1_W2, mlp1_b2,
           fc1_W, fc1_b, fc2_W, fc2_b):
    src = edge_index[0]
    dst = edge_index[1]
    e0, e1 = _edge_linear_both(edge_attr.T, lin0_W, lin0_b, lin1_W, lin1_b)
    p0 = _sc_partial_agg(x, e0, src, dst)
    h1 = _mlp_layer(x, p0, mlp0_W1, mlp0_b1, mlp0_W2, mlp0_b2)
    p1 = _sc_partial_agg(h1, e1, src, dst)
    return _mlp_final(h1, p1, mlp1_W1, mlp1_b1, mlp1_W2, mlp1_b2,
                      fc1_W, fc1_b, fc2_W, fc2_b)


# speedup vs baseline: 1.0271x; 1.0271x over previous
import functools

import jax
import jax.numpy as jnp
from jax import lax
from jax.experimental import pallas as pl
from jax.experimental.pallas import tpu as pltpu
from jax.experimental.pallas import tpu_sc as plsc

N = 10000
E = 320000
D = 128
D_EDGE = 16

NC = 2
NS = 16
LANES = 16

C = 40
EPT = E // (NC * NS)
NCH = EPT // C
PASS_NCH = (124, 126)
SLAB = 126 * C
RQ = 624
TAIL = N - NS * RQ


def _edge_linear(ea_T, W_lin, b):
    BE = 16000
    dn = (((0,), (0,)), ((), ()))

    def body(ea_ref, w_ref, b_ref, o_ref):
        o_ref[...] = (
            lax.dot_general(
                ea_ref[...], w_ref[...], dn, preferred_element_type=jnp.float32
            )
            + b_ref[...]
        )

    return pl.pallas_call(
        body,
        grid=(E // BE,),
        in_specs=[
            pl.BlockSpec((D_EDGE, BE), lambda i: (0, i)),
            pl.BlockSpec((D_EDGE, D), lambda i: (0, 0)),
            pl.BlockSpec((1, D), lambda i: (0, 0)),
        ],
        out_specs=pl.BlockSpec((BE, D), lambda i: (i, 0)),
        out_shape=jax.ShapeDtypeStruct((E, D), jnp.float32),
    )(ea_T, W_lin, b.reshape(1, D))


def _sc_partial_agg(h, e, src, dst):
    mesh = plsc.VectorSubcoreMesh(
        core_axis_name="c", subcore_axis_name="s", num_cores=NC, num_subcores=NS
    )

    @functools.partial(
        pl.kernel,
        out_type=jax.ShapeDtypeStruct((NC, N, D), jnp.float32),
        mesh=mesh,
        scratch_types=[
            pltpu.VMEM((SLAB,), jnp.int32),
            pltpu.VMEM((SLAB,), jnp.int32),
            pltpu.VMEM((C, D), jnp.float32),
            pltpu.VMEM((C, D), jnp.float32),
            pltpu.VMEM((C, D), jnp.float32),
            pltpu.VMEM((C, D), jnp.float32),
            pltpu.VMEM((C, D), jnp.float32),
            pltpu.VMEM((C, D), jnp.float32),
            pltpu.VMEM((C,), jnp.int32),
            pltpu.VMEM((C,), jnp.int32),
            pltpu.VMEM_SHARED((N, D), jnp.float32),
            pltpu.SemaphoreType.DMA,
            pltpu.SemaphoreType.DMA,
            pltpu.SemaphoreType.DMA,
            pltpu.SemaphoreType.DMA,
            pltpu.SemaphoreType.DMA,
            pltpu.SemaphoreType.DMA,
            pltpu.SemaphoreType.DMA,
        ],
    )
    def k(h_hbm, e_hbm, src_hbm, dst_hbm, out_hbm, src_v, dst_v,
          g0, g1, e0, e1, m0, m1, du0, du1, agg_s,
          isem, gs0, gs1, es0, es1, ss0, ss1):
        cid = lax.axis_index("c")
        sid = lax.axis_index("s")
        wid = sid * NC + cid
        base_e = wid * EPT

        bufs = ((g0, e0, m0, du0, gs0, es0, ss0),
                (g1, e1, m1, du1, gs1, es1, ss1))

        def load_slabs(eoff, n_edges):
            pltpu.async_copy(
                src_hbm.at[pl.ds(eoff, n_edges)], src_v.at[pl.ds(0, n_edges)], isem
            )
            pltpu.async_copy(
                dst_hbm.at[pl.ds(eoff, n_edges)], dst_v.at[pl.ds(0, n_edges)], isem
            )

        def wait_slabs(eoff, n_edges):
            pltpu.make_async_copy(
                src_hbm.at[pl.ds(eoff, n_edges)], src_v.at[pl.ds(0, n_edges)], isem
            ).wait()
            pltpu.make_async_copy(
                dst_hbm.at[pl.ds(eoff, n_edges)], dst_v.at[pl.ds(0, n_edges)], isem
            ).wait()

        load_slabs(base_e, PASS_NCH[0] * C)

        @pl.loop(0, C)
        def _(r):
            for j in range(0, D, LANES):
                m0[r, pl.ds(j, LANES)] = jnp.zeros((LANES,), jnp.float32)

        @pl.loop(0, RQ - C + 1, step=C)
        def _(r0):
            pltpu.sync_copy(m0, agg_s.at[pl.ds(sid * RQ + r0, C)])

        pltpu.sync_copy(
            m0.at[pl.ds(0, RQ - (RQ // C) * C)],
            agg_s.at[pl.ds(sid * RQ + (RQ // C) * C, RQ - (RQ // C) * C)],
        )

        @pl.when(sid == 0)
        def _():
            pltpu.sync_copy(m0.at[pl.ds(0, TAIL)], agg_s.at[pl.ds(NS * RQ, TAIL)])

        wait_slabs(base_e, PASS_NCH[0] * C)
        plsc.subcore_barrier()

        def wait_ge(b):
            g, ev, _, _, gs, es, _ = bufs[b]
            pltpu.make_async_copy(h_hbm.at[pl.ds(0, C)], g, gs).wait()
            pltpu.make_async_copy(h_hbm.at[pl.ds(0, C)], ev, es).wait()

        def wait_s(b):
            _, _, m, _, _, _, ss = bufs[b]
            pltpu.make_async_copy(h_hbm.at[pl.ds(0, C)], m, ss).wait()

        def compute(b):
            g, ev, m, _, _, _, _ = bufs[b]

            @plsc.parallel_loop(0, C, step=1, unroll=4)
            def _(r):
                for j in range(0, D, LANES):
                    m[r, pl.ds(j, LANES)] = jnp.maximum(
                        g[r, pl.ds(j, LANES)] + ev[r, pl.ds(j, LANES)], 0.0
                    )

        def run_pass(eoff, nch):
            def issue(i, b):
                g, ev, _, _, gs, es, _ = bufs[b]
                pltpu.async_copy(h_hbm.at[src_v.at[pl.ds(i * C, C)]], g, gs)
                pltpu.async_copy(e_hbm.at[pl.ds(eoff + i * C, C)], ev, es)

            def scatter(i, b):
                _, _, m, du, _, _, ss = bufs[b]
                for kk in (0, 16, 24):
                    du[pl.ds(kk, LANES)] = dst_v[pl.ds(i * C + kk, LANES)]
                pltpu.async_copy(m, agg_s.at[du], ss, add=True)

            issue(0, 0)
            issue(1, 1)

            wait_ge(0)
            compute(0)
            issue(2, 0)
            scatter(0, 0)
            wait_ge(1)
            compute(1)
            issue(3, 1)
            scatter(1, 1)

            @pl.loop(2, nch - 2, step=2)
            def _(i):
                wait_ge(0)
                wait_s(0)
                compute(0)
                issue(i + 2, 0)
                scatter(i, 0)
                wait_ge(1)
                wait_s(1)
                compute(1)
                issue(i + 3, 1)
                scatter(i + 1, 1)

            wait_ge(0)
            wait_s(0)
            compute(0)
            scatter(nch - 2, 0)
            wait_ge(1)
            wait_s(1)
            compute(1)
            scatter(nch - 1, 1)
            wait_s(0)
            wait_s(1)

        run_pass(base_e, PASS_NCH[0])
        load_slabs(base_e + PASS_NCH[0] * C, PASS_NCH[1] * C)
        wait_slabs(base_e + PASS_NCH[0] * C, PASS_NCH[1] * C)
        run_pass(base_e + PASS_NCH[0] * C, PASS_NCH[1])

        plsc.subcore_barrier()
        pltpu.sync_copy(
            agg_s.at[pl.ds(sid * RQ, RQ)],
            out_hbm.at[cid, pl.ds(sid * RQ, RQ)],
        )

        @pl.when(sid == 0)
        def _():
            pltpu.sync_copy(
                agg_s.at[pl.ds(NS * RQ, TAIL)],
                out_hbm.at[cid, pl.ds(NS * RQ, TAIL)],
            )

    return k(h, e, src, dst)


def _mlp_layer(h, p, W1, b1, W2, b2):
    BR = 2000

    def body(h_ref, p_ref, w1, b1r, w2, b2r, o_ref):
        z = h_ref[...] + p_ref[0] + p_ref[1]
        t = jnp.maximum(
            jnp.dot(z, w1[...], preferred_element_type=jnp.float32) + b1r[...], 0.0
        )
        o_ref[...] = jnp.maximum(
            jnp.dot(t, w2[...], preferred_element_type=jnp.float32) + b2r[...], 0.0
        )

    return pl.pallas_call(
        body,
        grid=(N // BR,),
        in_specs=[
            pl.BlockSpec((BR, D), lambda i: (i, 0)),
            pl.BlockSpec((NC, BR, D), lambda i: (0, i, 0)),
            pl.BlockSpec((D, D), lambda i: (0, 0)),
            pl.BlockSpec((1, D), lambda i: (0, 0)),
            pl.BlockSpec((D, D), lambda i: (0, 0)),
            pl.BlockSpec((1, D), lambda i: (0, 0)),
        ],
        out_specs=pl.BlockSpec((BR, D), lambda i: (i, 0)),
        out_shape=jax.ShapeDtypeStruct((N, D), jnp.float32),
    )(h, p, W1, b1.reshape(1, D), W2, b2.reshape(1, D))


def _mlp_final(h, p, W1, b1, W2, b2, fc1_W, fc1_b, fc2_W, fc2_b):
    BR = 2000

    def body(h_ref, p_ref, w1, b1r, w2, b2r, f1, f1b, f2, f2b, o_ref):
        z = h_ref[...] + p_ref[0] + p_ref[1]
        t = jnp.maximum(
            jnp.dot(z, w1[...], preferred_element_type=jnp.float32) + b1r[...], 0.0
        )
        h2 = jnp.maximum(
            jnp.dot(t, w2[...], preferred_element_type=jnp.float32) + b2r[...], 0.0
        )
        t2 = jnp.maximum(
            jnp.dot(h2, f1[...], preferred_element_type=jnp.float32) + f1b[...], 0.0
        )
        o_ref[...] = (
            jnp.dot(t2, f2[...], preferred_element_type=jnp.float32) + f2b[...]
        )

    return pl.pallas_call(
        body,
        grid=(N // BR,),
        in_specs=[
            pl.BlockSpec((BR, D), lambda i: (i, 0)),
            pl.BlockSpec((NC, BR, D), lambda i: (0, i, 0)),
            pl.BlockSpec((D, D), lambda i: (0, 0)),
            pl.BlockSpec((1, D), lambda i: (0, 0)),
            pl.BlockSpec((D, D), lambda i: (0, 0)),
            pl.BlockSpec((1, D), lambda i: (0, 0)),
            pl.BlockSpec((D, D), lambda i: (0, 0)),
            pl.BlockSpec((1, D), lambda i: (0, 0)),
            pl.BlockSpec((D, D), lambda i: (0, 0)),
            pl.BlockSpec((1, D), lambda i: (0, 0)),
        ],
        out_specs=pl.BlockSpec((BR, D), lambda i: (i, 0)),
        out_shape=jax.ShapeDtypeStruct((N, D), jnp.float32),
    )(h, p, W1, b1.reshape(1, D), W2, b2.reshape(1, D),
      fc1_W, fc1_b.reshape(1, D), fc2_W, fc2_b.reshape(1, D))


def kernel(x, edge_index, edge_attr,
           lin0_W, lin0_b, mlp0_W1, mlp0_b1, mlp0_W2, mlp0_b2,
           lin1_W, lin1_b, mlp1_W1, mlp1_b1, mlp1_W2, mlp1_b2,
           fc1_W, fc1_b, fc2_W, fc2_b):
    src = edge_index[0]
    dst = edge_index[1]
    ea_T = edge_attr.T
    e0 = _edge_linear(ea_T, lin0_W, lin0_b)
    e1 = _edge_linear(ea_T, lin1_W, lin1_b)
    p0 = _sc_partial_agg(x, e0, src, dst)
    h1 = _mlp_layer(x, p0, mlp0_W1, mlp0_b1, mlp0_W2, mlp0_b2)
    p1 = _sc_partial_agg(h1, e1, src, dst)
    return _mlp_final(h1, p1, mlp1_W1, mlp1_b1, mlp1_W2, mlp1_b2,
                      fc1_W, fc1_b, fc2_W, fc2_b)


# speedup vs baseline: 1.0327x; 1.0055x over previous
import functools

import jax
import jax.numpy as jnp
from jax import lax
from jax.experimental import pallas as pl
from jax.experimental.pallas import tpu as pltpu
from jax.experimental.pallas import tpu_sc as plsc

N = 10000
E = 320000
D = 128
D_EDGE = 16

NC = 2
NS = 16
LANES = 16

C = 48
EPT = E // (NC * NS)
PASS_NCH = (104, 104)
ETAIL = EPT - (PASS_NCH[0] + PASS_NCH[1]) * C
SLAB = 104 * C
RQ = 624
TAIL = N - NS * RQ


def _edge_linear(ea_T, W_lin, b):
    BE = 16000
    dn = (((0,), (0,)), ((), ()))

    def body(ea_ref, w_ref, b_ref, o_ref):
        o_ref[...] = (
            lax.dot_general(
                ea_ref[...], w_ref[...], dn, preferred_element_type=jnp.float32
            )
            + b_ref[...]
        )

    return pl.pallas_call(
        body,
        grid=(E // BE,),
        in_specs=[
            pl.BlockSpec((D_EDGE, BE), lambda i: (0, i)),
            pl.BlockSpec((D_EDGE, D), lambda i: (0, 0)),
            pl.BlockSpec((1, D), lambda i: (0, 0)),
        ],
        out_specs=pl.BlockSpec((BE, D), lambda i: (i, 0)),
        out_shape=jax.ShapeDtypeStruct((E, D), jnp.float32),
    )(ea_T, W_lin, b.reshape(1, D))


def _sc_partial_agg(h, e, src, dst):
    mesh = plsc.VectorSubcoreMesh(
        core_axis_name="c", subcore_axis_name="s", num_cores=NC, num_subcores=NS
    )

    @functools.partial(
        pl.kernel,
        out_type=jax.ShapeDtypeStruct((NC, N, D), jnp.float32),
        mesh=mesh,
        scratch_types=[
            pltpu.VMEM((SLAB,), jnp.int32),
            pltpu.VMEM((SLAB,), jnp.int32),
            pltpu.VMEM((C, D), jnp.float32),
            pltpu.VMEM((C, D), jnp.float32),
            pltpu.VMEM((C, D), jnp.float32),
            pltpu.VMEM((C, D), jnp.float32),
            pltpu.VMEM((C, D), jnp.float32),
            pltpu.VMEM((C, D), jnp.float32),
            pltpu.VMEM((C,), jnp.int32),
            pltpu.VMEM((C,), jnp.int32),
            pltpu.VMEM((ETAIL,), jnp.int32),
            pltpu.VMEM((ETAIL,), jnp.int32),
            pltpu.VMEM_SHARED((N, D), jnp.float32),
            pltpu.SemaphoreType.DMA,
            pltpu.SemaphoreType.DMA,
            pltpu.SemaphoreType.DMA,
            pltpu.SemaphoreType.DMA,
            pltpu.SemaphoreType.DMA,
            pltpu.SemaphoreType.DMA,
            pltpu.SemaphoreType.DMA,
        ],
    )
    def k(h_hbm, e_hbm, src_hbm, dst_hbm, out_hbm, src_v, dst_v,
          g0, g1, e0, e1, m0, m1, du0, du1, st_src, st_dst, agg_s,
          isem, gs0, gs1, es0, es1, ss0, ss1):
        cid = lax.axis_index("c")
        sid = lax.axis_index("s")
        wid = sid * NC + cid
        base_e = wid * EPT

        bufs = ((g0, e0, m0, du0, gs0, es0, ss0),
                (g1, e1, m1, du1, gs1, es1, ss1))

        def load_slabs(eoff, n_edges):
            pltpu.async_copy(
                src_hbm.at[pl.ds(eoff, n_edges)], src_v.at[pl.ds(0, n_edges)], isem
            )
            pltpu.async_copy(
                dst_hbm.at[pl.ds(eoff, n_edges)], dst_v.at[pl.ds(0, n_edges)], isem
            )

        def wait_slabs(eoff, n_edges):
            pltpu.make_async_copy(
                src_hbm.at[pl.ds(eoff, n_edges)], src_v.at[pl.ds(0, n_edges)], isem
            ).wait()
            pltpu.make_async_copy(
                dst_hbm.at[pl.ds(eoff, n_edges)], dst_v.at[pl.ds(0, n_edges)], isem
            ).wait()

        load_slabs(base_e, PASS_NCH[0] * C)

        @pl.loop(0, C)
        def _(r):
            for j in range(0, D, LANES):
                m0[r, pl.ds(j, LANES)] = jnp.zeros((LANES,), jnp.float32)

        @pl.loop(0, RQ - C + 1, step=C)
        def _(r0):
            pltpu.sync_copy(m0, agg_s.at[pl.ds(sid * RQ + r0, C)])

        @pl.when(sid == 0)
        def _():
            pltpu.sync_copy(m0.at[pl.ds(0, TAIL)], agg_s.at[pl.ds(NS * RQ, TAIL)])

        wait_slabs(base_e, PASS_NCH[0] * C)
        plsc.subcore_barrier()

        tail_off = base_e + (PASS_NCH[0] + PASS_NCH[1]) * C
        pltpu.sync_copy(src_hbm.at[pl.ds(tail_off, ETAIL)], st_src)
        pltpu.sync_copy(dst_hbm.at[pl.ds(tail_off, ETAIL)], st_dst)
        pltpu.async_copy(h_hbm.at[st_src], g0.at[pl.ds(0, ETAIL)], gs0).wait()
        pltpu.sync_copy(e_hbm.at[pl.ds(tail_off, ETAIL)], e0.at[pl.ds(0, ETAIL)])

        @plsc.parallel_loop(0, ETAIL, step=1, unroll=4)
        def _(r):
            for j in range(0, D, LANES):
                m0[r, pl.ds(j, LANES)] = jnp.maximum(
                    g0[r, pl.ds(j, LANES)] + e0[r, pl.ds(j, LANES)], 0.0
                )

        pltpu.sync_copy(m0.at[pl.ds(0, ETAIL)], agg_s.at[st_dst], add=True)

        def wait_ge(b):
            g, ev, _, _, gs, es, _ = bufs[b]
            pltpu.make_async_copy(h_hbm.at[pl.ds(0, C)], g, gs).wait()
            pltpu.make_async_copy(h_hbm.at[pl.ds(0, C)], ev, es).wait()

        def wait_s(b):
            _, _, m, _, _, _, ss = bufs[b]
            pltpu.make_async_copy(h_hbm.at[pl.ds(0, C)], m, ss).wait()

        def compute(b):
            g, ev, m, _, _, _, _ = bufs[b]

            @plsc.parallel_loop(0, C, step=1, unroll=4)
            def _(r):
                for j in range(0, D, LANES):
                    m[r, pl.ds(j, LANES)] = jnp.maximum(
                        g[r, pl.ds(j, LANES)] + ev[r, pl.ds(j, LANES)], 0.0
                    )

        def run_pass(eoff, nch):
            def issue(i, b):
                g, ev, _, _, gs, es, _ = bufs[b]
                pltpu.async_copy(h_hbm.at[src_v.at[pl.ds(i * C, C)]], g, gs)
                pltpu.async_copy(e_hbm.at[pl.ds(eoff + i * C, C)], ev, es)

            def scatter(i, b):
                _, _, m, du, _, _, ss = bufs[b]
                for kk in range(0, C, LANES):
                    du[pl.ds(kk, LANES)] = dst_v[pl.ds(i * C + kk, LANES)]
                pltpu.async_copy(m, agg_s.at[du], ss, add=True)

            issue(0, 0)
            issue(1, 1)

            wait_ge(0)
            compute(0)
            issue(2, 0)
            scatter(0, 0)
            wait_ge(1)
            compute(1)
            issue(3, 1)
            scatter(1, 1)

            @pl.loop(2, nch - 2, step=2)
            def _(i):
                wait_ge(0)
                wait_s(0)
                compute(0)
                issue(i + 2, 0)
                scatter(i, 0)
                wait_ge(1)
                wait_s(1)
                compute(1)
                issue(i + 3, 1)
                scatter(i + 1, 1)

            wait_ge(0)
            wait_s(0)
            compute(0)
            scatter(nch - 2, 0)
            wait_ge(1)
            wait_s(1)
            compute(1)
            scatter(nch - 1, 1)
            wait_s(0)
            wait_s(1)

        run_pass(base_e, PASS_NCH[0])
        load_slabs(base_e + PASS_NCH[0] * C, PASS_NCH[1] * C)
        wait_slabs(base_e + PASS_NCH[0] * C, PASS_NCH[1] * C)
        run_pass(base_e + PASS_NCH[0] * C, PASS_NCH[1])

        plsc.subcore_barrier()
        pltpu.sync_copy(
            agg_s.at[pl.ds(sid * RQ, RQ)],
            out_hbm.at[cid, pl.ds(sid * RQ, RQ)],
        )

        @pl.when(sid == 0)
        def _():
            pltpu.sync_copy(
                agg_s.at[pl.ds(NS * RQ, TAIL)],
                out_hbm.at[cid, pl.ds(NS * RQ, TAIL)],
            )

    return k(h, e, src, dst)


def _mlp_layer(h, p, W1, b1, W2, b2):
    BR = 2000

    def body(h_ref, p_ref, w1, b1r, w2, b2r, o_ref):
        z = h_ref[...] + p_ref[0] + p_ref[1]
        t = jnp.maximum(
            jnp.dot(z, w1[...], preferred_element_type=jnp.float32) + b1r[...], 0.0
        )
        o_ref[...] = jnp.maximum(
            jnp.dot(t, w2[...], preferred_element_type=jnp.float32) + b2r[...], 0.0
        )

    return pl.pallas_call(
        body,
        grid=(N // BR,),
        in_specs=[
            pl.BlockSpec((BR, D), lambda i: (i, 0)),
            pl.BlockSpec((NC, BR, D), lambda i: (0, i, 0)),
            pl.BlockSpec((D, D), lambda i: (0, 0)),
            pl.BlockSpec((1, D), lambda i: (0, 0)),
            pl.BlockSpec((D, D), lambda i: (0, 0)),
            pl.BlockSpec((1, D), lambda i: (0, 0)),
        ],
        out_specs=pl.BlockSpec((BR, D), lambda i: (i, 0)),
        out_shape=jax.ShapeDtypeStruct((N, D), jnp.float32),
    )(h, p, W1, b1.reshape(1, D), W2, b2.reshape(1, D))


def _mlp_final(h, p, W1, b1, W2, b2, fc1_W, fc1_b, fc2_W, fc2_b):
    BR = 2000

    def body(h_ref, p_ref, w1, b1r, w2, b2r, f1, f1b, f2, f2b, o_ref):
        z = h_ref[...] + p_ref[0] + p_ref[1]
        t = jnp.maximum(
            jnp.dot(z, w1[...], preferred_element_type=jnp.float32) + b1r[...], 0.0
        )
        h2 = jnp.maximum(
            jnp.dot(t, w2[...], preferred_element_type=jnp.float32) + b2r[...], 0.0
        )
        t2 = jnp.maximum(
            jnp.dot(h2, f1[...], preferred_element_type=jnp.float32) + f1b[...], 0.0
        )
        o_ref[...] = (
            jnp.dot(t2, f2[...], preferred_element_type=jnp.float32) + f2b[...]
        )

    return pl.pallas_call(
        body,
        grid=(N // BR,),
        in_specs=[
            pl.BlockSpec((BR, D), lambda i: (i, 0)),
            pl.BlockSpec((NC, BR, D), lambda i: (0, i, 0)),
            pl.BlockSpec((D, D), lambda i: (0, 0)),
            pl.BlockSpec((1, D), lambda i: (0, 0)),
            pl.BlockSpec((D, D), lambda i: (0, 0)),
            pl.BlockSpec((1, D), lambda i: (0, 0)),
            pl.BlockSpec((D, D), lambda i: (0, 0)),
            pl.BlockSpec((1, D), lambda i: (0, 0)),
            pl.BlockSpec((D, D), lambda i: (0, 0)),
            pl.BlockSpec((1, D), lambda i: (0, 0)),
        ],
        out_specs=pl.BlockSpec((BR, D), lambda i: (i, 0)),
        out_shape=jax.ShapeDtypeStruct((N, D), jnp.float32),
    )(h, p, W1, b1.reshape(1, D), W2, b2.reshape(1, D),
      fc1_W, fc1_b.reshape(1, D), fc2_W, fc2_b.reshape(1, D))


def kernel(x, edge_index, edge_attr,
           lin0_W, lin0_b, mlp0_W1, mlp0_b1, mlp0_W2, mlp0_b2,
           lin1_W, lin1_b, mlp1_W1, mlp1_b1, mlp1_W2, mlp1_b2,
           fc1_W, fc1_b, fc2_W, fc2_b):
    src = edge_index[0]
    dst = edge_index[1]
    ea_T = edge_attr.T
    e0 = _edge_linear(ea_T, lin0_W, lin0_b)
    e1 = _edge_linear(ea_T, lin1_W, lin1_b)
    p0 = _sc_partial_agg(x, e0, src, dst)
    h1 = _mlp_layer(x, p0, mlp0_W1, mlp0_b1, mlp0_W2, mlp0_b2)
    p1 = _sc_partial_agg(h1, e1, src, dst)
    return _mlp_final(h1, p1, mlp1_W1, mlp1_b1, mlp1_W2, mlp1_b2,
                      fc1_W, fc1_b, fc2_W, fc2_b)
